# baseline (speedup 1.0000x reference)
import jax
import jax.numpy as jnp
from jax.experimental import pallas as pl
from jax.experimental.pallas import tpu as pltpu

S = 512
BLOCK = 512


def _net_kernel(x_ref, wv1_ref, a_ref, bq1_ref, wq2_ref, bq2_ref,
                b_ref, bvh1_ref, wvh2_ref, bvh2_ref, out_ref):
    s = x_ref[...]
    h = jnp.maximum(jnp.dot(s, wv1_ref[...], preferred_element_type=jnp.float32), 0.0)
    hq = jnp.maximum(
        jnp.dot(h, a_ref[...], preferred_element_type=jnp.float32) + bq1_ref[...], 0.0)
    hv = jnp.maximum(
        jnp.dot(h, b_ref[...], preferred_element_type=jnp.float32) + bvh1_ref[...], 0.0)
    q = jnp.dot(hq, wq2_ref[...], preferred_element_type=jnp.float32) + bq2_ref[...]
    vs = jnp.sum(hv * wvh2_ref[...], axis=1, keepdims=True) + bvh2_ref[...]
    out_ref[...] = q + vs


def kernel(x, Wq1, bq1, Wq2, bq2, Wk1, Wk2, Wv1, Wv2,
           WQ1, bQ1, WQ2, bQ2, WV1, bV1, WV2, bV2):
    ba = x.shape[0]
    wv1 = Wv1.T
    a = Wv2.T @ WQ1.T
    b = Wv2.T @ WV1.T
    wq2c = (WQ2 - jnp.mean(WQ2, axis=0, keepdims=True)).T
    bq2c = (bQ2 - jnp.mean(bQ2)).reshape(1, -1)
    bq1_ = bQ1.reshape(1, -1)
    bvh1_ = bV1.reshape(1, -1)
    wvh2 = WV2
    bvh2_ = bV2.reshape(1, -1)

    out_dim = WQ2.shape[0]
    grid = (ba // BLOCK,)

    def full(arr):
        return pl.BlockSpec(arr.shape, lambda i: (0,) * arr.ndim)

    return pl.pallas_call(
        _net_kernel,
        grid=grid,
        in_specs=[
            pl.BlockSpec((BLOCK, S), lambda i: (i, 0)),
            full(wv1), full(a), full(bq1_), full(wq2c), full(bq2c),
            full(b), full(bvh1_), full(wvh2), full(bvh2_),
        ],
        out_specs=pl.BlockSpec((BLOCK, out_dim), lambda i: (i, 0)),
        out_shape=jax.ShapeDtypeStruct((ba, out_dim), jnp.float32),
        compiler_params=pltpu.CompilerParams(
            dimension_semantics=("parallel",)),
    )(x, wv1, a, bq1_, wq2c, bq2c, b, bvh1_, wvh2, bvh2_)


# speedup vs baseline: 1.3522x; 1.3522x over previous
import jax
import jax.numpy as jnp
from jax.experimental import pallas as pl
from jax.experimental.pallas import tpu as pltpu

S = 512
BLOCK = 512

_DN_T = (((1,), (1,)), ((), ()))


def _mmt(a, w):
    return jax.lax.dot_general(a, w, _DN_T, preferred_element_type=jnp.float32)


def _net_kernel(x_ref, wv1_ref, wv2_ref, wq1_ref, bq1_ref, wq2_ref, bq2_ref,
                wvh1_ref, bvh1_ref, wvh2_ref, bvh2_ref, out_ref):
    s = x_ref[...]
    h = jnp.maximum(_mmt(s, wv1_ref[...]), 0.0)
    v = _mmt(h, wv2_ref[...])
    hq = jnp.maximum(_mmt(v, wq1_ref[...]) + bq1_ref[...], 0.0)
    q = _mmt(hq, wq2_ref[...]) + bq2_ref[...]
    hv = jnp.maximum(_mmt(v, wvh1_ref[...]) + bvh1_ref[...], 0.0)
    vs = jnp.sum(hv * wvh2_ref[...], axis=1, keepdims=True) + bvh2_ref[...]
    out_ref[...] = q - jnp.mean(q, axis=1, keepdims=True) + vs


def kernel(x, Wq1, bq1, Wq2, bq2, Wk1, Wk2, Wv1, Wv2,
           WQ1, bQ1, WQ2, bQ2, WV1, bV1, WV2, bV2):
    ba = x.shape[0]
    bq1_ = bQ1.reshape(1, -1)
    bq2_ = bQ2.reshape(1, -1)
    bvh1_ = bV1.reshape(1, -1)
    bvh2_ = bV2.reshape(1, -1)

    out_dim = WQ2.shape[0]
    grid = (ba // BLOCK,)

    def full(arr):
        return pl.BlockSpec(arr.shape, lambda i: (0,) * arr.ndim)

    return pl.pallas_call(
        _net_kernel,
        grid=grid,
        in_specs=[
            pl.BlockSpec((BLOCK, S), lambda i: (i, 0)),
            full(Wv1), full(Wv2), full(WQ1), full(bq1_), full(WQ2), full(bq2_),
            full(WV1), full(bvh1_), full(WV2), full(bvh2_),
        ],
        out_specs=pl.BlockSpec((BLOCK, out_dim), lambda i: (i, 0)),
        out_shape=jax.ShapeDtypeStruct((ba, out_dim), jnp.float32),
        compiler_params=pltpu.CompilerParams(
            dimension_semantics=("parallel",)),
    )(x, Wv1, Wv2, WQ1, bq1_, WQ2, bq2_, WV1, bvh1_, WV2, bvh2_)


# speedup vs baseline: 1.3529x; 1.0005x over previous
import jax
import jax.numpy as jnp
from jax.experimental import pallas as pl
from jax.experimental.pallas import tpu as pltpu

S = 512
BLOCK = 512

_DN_T = (((1,), (1,)), ((), ()))


def _mmt(a, w):
    return jax.lax.dot_general(a, w, _DN_T, preferred_element_type=jnp.float32)


def _net_kernel(x_ref, wv1_ref, wv2_ref, wq1_ref, bq1_ref, wq2_ref, bq2_ref,
                wvh1_ref, bvh1_ref, wvh2_ref, bvh2_ref, out_ref):
    s = x_ref[...]
    h = jnp.maximum(_mmt(s, wv1_ref[...]), 0.0)
    v = _mmt(h, wv2_ref[...])
    hq = jnp.maximum(_mmt(v, wq1_ref[...]) + bq1_ref[...], 0.0)
    q = _mmt(hq, wq2_ref[...]) + bq2_ref[...]
    hv = jnp.maximum(_mmt(v, wvh1_ref[...]) + bvh1_ref[...], 0.0)
    vs = jnp.sum(hv * wvh2_ref[...], axis=1, keepdims=True) + bvh2_ref[...]
    out_ref[...] = q - jnp.mean(q, axis=1, keepdims=True) + vs


def kernel(x, Wq1, bq1, Wq2, bq2, Wk1, Wk2, Wv1, Wv2,
           WQ1, bQ1, WQ2, bQ2, WV1, bV1, WV2, bV2):
    ba = x.shape[0]
    bq1_, bq2_, bvh1_, bvh2_ = bQ1, bQ2, bV1, bV2

    out_dim = WQ2.shape[0]
    grid = (ba // BLOCK,)

    def full(arr):
        return pl.BlockSpec(arr.shape, lambda i: (0,) * arr.ndim)

    return pl.pallas_call(
        _net_kernel,
        grid=grid,
        in_specs=[
            pl.BlockSpec((BLOCK, S), lambda i: (i, 0)),
            full(Wv1), full(Wv2), full(WQ1), full(bq1_), full(WQ2), full(bq2_),
            full(WV1), full(bvh1_), full(WV2), full(bvh2_),
        ],
        out_specs=pl.BlockSpec((BLOCK, out_dim), lambda i: (i, 0)),
        out_shape=jax.ShapeDtypeStruct((ba, out_dim), jnp.float32),
        compiler_params=pltpu.CompilerParams(
            dimension_semantics=("parallel",)),
    )(x, Wv1, Wv2, WQ1, bq1_, WQ2, bq2_, WV1, bvh1_, WV2, bvh2_)


# speedup vs baseline: 1.3697x; 1.0124x over previous
import jax
import jax.numpy as jnp
from jax.experimental import pallas as pl
from jax.experimental.pallas import tpu as pltpu

S = 512
BLOCK = 512

_DN_T = (((1,), (1,)), ((), ()))


def _mmt(a, w):
    return jax.lax.dot_general(a, w, _DN_T, preferred_element_type=jnp.float32)


SPLIT = 2


def _net_kernel(x_ref, wv1_ref, wv2_ref, wq1_ref, bq1_ref, wq2_ref, bq2_ref,
                wvh1_ref, bvh1_ref, wvh2_ref, bvh2_ref, out_ref):
    sub = BLOCK // SPLIT
    for p in range(SPLIT):
        rows = pl.ds(p * sub, sub)
        s = x_ref[rows, :]
        h = jnp.maximum(_mmt(s, wv1_ref[...]), 0.0)
        v = _mmt(h, wv2_ref[...])
        hq = jnp.maximum(_mmt(v, wq1_ref[...]) + bq1_ref[...], 0.0)
        q = _mmt(hq, wq2_ref[...]) + bq2_ref[...]
        hv = jnp.maximum(_mmt(v, wvh1_ref[...]) + bvh1_ref[...], 0.0)
        vs = jnp.sum(hv * wvh2_ref[...], axis=1, keepdims=True) + bvh2_ref[...]
        out_ref[rows, :] = q - jnp.mean(q, axis=1, keepdims=True) + vs


def kernel(x, Wq1, bq1, Wq2, bq2, Wk1, Wk2, Wv1, Wv2,
           WQ1, bQ1, WQ2, bQ2, WV1, bV1, WV2, bV2):
    ba = x.shape[0]
    bq1_, bq2_, bvh1_, bvh2_ = bQ1, bQ2, bV1, bV2

    out_dim = WQ2.shape[0]
    grid = (ba // BLOCK,)

    def full(arr):
        return pl.BlockSpec(arr.shape, lambda i: (0,) * arr.ndim)

    return pl.pallas_call(
        _net_kernel,
        grid=grid,
        in_specs=[
            pl.BlockSpec((BLOCK, S), lambda i: (i, 0)),
            full(Wv1), full(Wv2), full(WQ1), full(bq1_), full(WQ2), full(bq2_),
            full(WV1), full(bvh1_), full(WV2), full(bvh2_),
        ],
        out_specs=pl.BlockSpec((BLOCK, out_dim), lambda i: (i, 0)),
        out_shape=jax.ShapeDtypeStruct((ba, out_dim), jnp.float32),
        compiler_params=pltpu.CompilerParams(
            dimension_semantics=("parallel",)),
    )(x, Wv1, Wv2, WQ1, bq1_, WQ2, bq2_, WV1, bvh1_, WV2, bvh2_)
